# Initial kernel scaffold; baseline (speedup 1.0000x reference)
import functools

import jax
import jax.numpy as jnp
from jax import lax
from jax.experimental import pallas as pl
from jax.experimental.pallas import tpu as pltpu
from jax.experimental.pallas import tpu_sc as plsc

NC, NS, L = 2, 16, 16
NW = NC * NS
T = 64
EPS = 1e-8


def _make_sc_kernel(M, C, interpret=False):
    assert M % T == 0 and C % L == 0
    n_chunks = M // T
    mesh = plsc.VectorSubcoreMesh(core_axis_name="c", subcore_axis_name="s")

    @functools.partial(
        pl.kernel,
        out_type=jax.ShapeDtypeStruct((M, C), jnp.float32),
        mesh=mesh,
        interpret=interpret,
        scratch_types=[
            pltpu.VMEM((T,), jnp.int32),
            pltpu.VMEM((T,), jnp.int32),
            pltpu.VMEM((T,), jnp.int32),
            pltpu.VMEM((T,), jnp.float32),
            pltpu.VMEM((T,), jnp.float32),
            pltpu.VMEM((T,), jnp.float32),
            pltpu.VMEM((T,), jnp.float32),
            pltpu.VMEM((T,), jnp.float32),
            pltpu.VMEM((T,), jnp.float32),
            pltpu.VMEM((T, C), jnp.float32),
            pltpu.VMEM((T, C), jnp.float32),
            pltpu.VMEM((T, C), jnp.float32),
            pltpu.VMEM((T, C), jnp.float32),
            pltpu.SemaphoreType.DMA,
        ],
    )
    def k(x_h, i0_h, i1_h, i2_h, d0_h, d1_h, d2_h, out_h,
          iv0, iv1, iv2, dv0, dv1, dv2, wv0, wv1, wv2, r0, r1, r2, ov, sem):
        wid = lax.axis_index("s") * NC + lax.axis_index("c")
        nch = (n_chunks - wid + NW - 1) // NW

        def chunk_body(i, carry):
            base = (wid + i * NW) * T
            pltpu.sync_copy(i0_h.at[pl.ds(base, T)], iv0)
            pltpu.sync_copy(i1_h.at[pl.ds(base, T)], iv1)
            pltpu.sync_copy(i2_h.at[pl.ds(base, T)], iv2)
            cp0 = pltpu.async_copy(x_h.at[iv0], r0, sem)
            cp1 = pltpu.async_copy(x_h.at[iv1], r1, sem)
            cp2 = pltpu.async_copy(x_h.at[iv2], r2, sem)
            pltpu.sync_copy(d0_h.at[pl.ds(base, T)], dv0)
            pltpu.sync_copy(d1_h.at[pl.ds(base, T)], dv1)
            pltpu.sync_copy(d2_h.at[pl.ds(base, T)], dv2)
            for j in range(T // L):
                sl = pl.ds(j * L, L)
                q0 = 1.0 / (dv0[sl] + EPS)
                q1 = 1.0 / (dv1[sl] + EPS)
                q2 = 1.0 / (dv2[sl] + EPS)
                nrm = q0 + q1 + q2
                wv0[sl] = q0 / nrm
                wv1[sl] = q1 / nrm
                wv2[sl] = q2 / nrm
            cp0.wait()
            cp1.wait()
            cp2.wait()

            def tgt(t, c2):
                tt = jnp.full((L,), t, jnp.int32)
                b0 = plsc.load_gather(wv0, [tt])
                b1 = plsc.load_gather(wv1, [tt])
                b2 = plsc.load_gather(wv2, [tt])
                for kk in range(C // L):
                    s = pl.ds(kk * L, L)
                    ov[t, s] = b0 * r0[t, s] + b1 * r1[t, s] + b2 * r2[t, s]
                return c2

            lax.fori_loop(0, T, tgt, 0)
            pltpu.sync_copy(ov, out_h.at[pl.ds(base, T)])
            return carry

        lax.fori_loop(0, nch, chunk_body, 0)

    return k


def kernel(x, upsample_inds, upsample_dists):
    M = upsample_inds.shape[0]
    C = x.shape[1]
    inds = upsample_inds.astype(jnp.int32)
    d = upsample_dists.astype(jnp.float32)
    sc = _make_sc_kernel(M, C)
    return sc(x, inds[:, 0], inds[:, 1], inds[:, 2], d[:, 0], d[:, 1], d[:, 2])


# speedup vs baseline: 6.7052x; 6.7052x over previous
import functools

import jax
import jax.numpy as jnp
from jax import lax
from jax.experimental import pallas as pl
from jax.experimental.pallas import tpu as pltpu
from jax.experimental.pallas import tpu_sc as plsc

NC, NS, L = 2, 16, 16
NW = NC * NS
T = 64
EPS = 1e-8


def _make_sc_kernel(M, C, interpret=False):
    assert M % T == 0 and C % L == 0
    n_chunks = M // T
    mesh = plsc.VectorSubcoreMesh(
        core_axis_name="c", subcore_axis_name="s",
        num_cores=NC, num_subcores=NS)

    @functools.partial(
        pl.kernel,
        out_type=jax.ShapeDtypeStruct((M, C), jnp.float32),
        mesh=mesh,
        interpret=interpret,
        scratch_types=[
            pltpu.VMEM((T,), jnp.int32),
            pltpu.VMEM((T,), jnp.int32),
            pltpu.VMEM((T,), jnp.int32),
            pltpu.VMEM((T,), jnp.float32),
            pltpu.VMEM((T,), jnp.float32),
            pltpu.VMEM((T,), jnp.float32),
            pltpu.VMEM((T,), jnp.float32),
            pltpu.VMEM((T,), jnp.float32),
            pltpu.VMEM((T,), jnp.float32),
            pltpu.VMEM((T, C), jnp.float32),
            pltpu.VMEM((T, C), jnp.float32),
            pltpu.VMEM((T, C), jnp.float32),
            pltpu.VMEM((T, C), jnp.float32),
            pltpu.SemaphoreType.DMA,
        ],
    )
    def k(x_h, i0_h, i1_h, i2_h, d0_h, d1_h, d2_h, out_h,
          iv0, iv1, iv2, dv0, dv1, dv2, wv0, wv1, wv2, r0, r1, r2, ov, sem):
        wid = lax.axis_index("s") * NC + lax.axis_index("c")
        nch = (n_chunks - wid + NW - 1) // NW

        def chunk_body(i, carry):
            base = (wid + i * NW) * T
            pltpu.sync_copy(i0_h.at[pl.ds(base, T)], iv0)
            pltpu.sync_copy(i1_h.at[pl.ds(base, T)], iv1)
            pltpu.sync_copy(i2_h.at[pl.ds(base, T)], iv2)
            cp0 = pltpu.async_copy(x_h.at[iv0], r0, sem)
            cp1 = pltpu.async_copy(x_h.at[iv1], r1, sem)
            cp2 = pltpu.async_copy(x_h.at[iv2], r2, sem)
            pltpu.sync_copy(d0_h.at[pl.ds(base, T)], dv0)
            pltpu.sync_copy(d1_h.at[pl.ds(base, T)], dv1)
            pltpu.sync_copy(d2_h.at[pl.ds(base, T)], dv2)
            for j in range(T // L):
                sl = pl.ds(j * L, L)
                q0 = 1.0 / (dv0[sl] + EPS)
                q1 = 1.0 / (dv1[sl] + EPS)
                q2 = 1.0 / (dv2[sl] + EPS)
                nrm = q0 + q1 + q2
                wv0[sl] = q0 / nrm
                wv1[sl] = q1 / nrm
                wv2[sl] = q2 / nrm
            cp0.wait()
            cp1.wait()
            cp2.wait()

            def tgt(g, c2):
                w0g = wv0[pl.ds(g * L, L)]
                w1g = wv1[pl.ds(g * L, L)]
                w2g = wv2[pl.ds(g * L, L)]
                for j in range(L):
                    t = g * L + j
                    b0 = jnp.full((L,), w0g[j], jnp.float32)
                    b1 = jnp.full((L,), w1g[j], jnp.float32)
                    b2 = jnp.full((L,), w2g[j], jnp.float32)
                    for kk in range(C // L):
                        s = pl.ds(kk * L, L)
                        ov[t, s] = (b0 * r0[t, s] + b1 * r1[t, s]
                                    + b2 * r2[t, s])
                return c2

            lax.fori_loop(0, T // L, tgt, 0)
            pltpu.sync_copy(ov, out_h.at[pl.ds(base, T)])
            return carry

        lax.fori_loop(0, nch, chunk_body, 0)

    return k


def kernel(x, upsample_inds, upsample_dists):
    M = upsample_inds.shape[0]
    C = x.shape[1]
    inds = upsample_inds.astype(jnp.int32)
    d = upsample_dists.astype(jnp.float32)
    sc = _make_sc_kernel(M, C)
    return sc(x, inds[:, 0], inds[:, 1], inds[:, 2], d[:, 0], d[:, 1], d[:, 2])


# speedup vs baseline: 13.3710x; 1.9941x over previous
import functools

import jax
import jax.numpy as jnp
from jax import lax
from jax.experimental import pallas as pl
from jax.experimental.pallas import tpu as pltpu
from jax.experimental.pallas import tpu_sc as plsc

NC, NS, L = 2, 16, 16
NW = NC * NS
T = 64
EPS = 1e-8


def _make_sc_kernel(M, C, interpret=False):
    assert M % T == 0 and C % L == 0
    n_chunks = M // T
    spw = (n_chunks + NW - 1) // NW
    assert spw % 2 == 0 or spw == 1
    mesh = plsc.VectorSubcoreMesh(
        core_axis_name="c", subcore_axis_name="s",
        num_cores=NC, num_subcores=NS)

    @functools.partial(
        pl.kernel,
        out_type=jax.ShapeDtypeStruct((M, C), jnp.float32),
        mesh=mesh,
        interpret=interpret,
        scratch_types=[
            [pltpu.VMEM((T,), jnp.int32) for _ in range(3)],
            [pltpu.VMEM((T,), jnp.int32) for _ in range(3)],
            [pltpu.VMEM((T,), jnp.float32) for _ in range(3)],
            [pltpu.VMEM((T,), jnp.float32) for _ in range(3)],
            [pltpu.VMEM((T,), jnp.float32) for _ in range(3)],
            [pltpu.VMEM((T,), jnp.float32) for _ in range(3)],
            [pltpu.VMEM((T, C), jnp.float32) for _ in range(3)],
            [pltpu.VMEM((T, C), jnp.float32) for _ in range(3)],
            pltpu.VMEM((T, C), jnp.float32),
            pltpu.VMEM((T, C), jnp.float32),
            [pltpu.SemaphoreType.DMA for _ in range(2)],
            [pltpu.SemaphoreType.DMA for _ in range(2)],
            [pltpu.SemaphoreType.DMA for _ in range(2)],
        ],
    )
    def k(x_h, i0_h, i1_h, i2_h, d0_h, d1_h, d2_h, out_h,
          iv0, iv1, dvv0, dvv1, wvv0, wvv1, rr0, rr1, ov0, ov1,
          sem_idx, sem_g, sem_out):
        iv = (iv0, iv1)
        dv = (dvv0, dvv1)
        wv = (wvv0, wvv1)
        rr = (rr0, rr1)
        ov = (ov0, ov1)
        ih = (i0_h, i1_h, i2_h)
        dh = (d0_h, d1_h, d2_h)
        wid = lax.axis_index("s") * NC + lax.axis_index("c")
        slot0 = wid * spw
        last = n_chunks - 1

        def cbase(slot):
            return jnp.minimum(slot, last) * T

        def fire_idx(slot, b):
            base = cbase(slot)
            for h in range(3):
                pltpu.async_copy(ih[h].at[pl.ds(base, T)], iv[b][h],
                                 sem_idx[b])
                pltpu.async_copy(dh[h].at[pl.ds(base, T)], dv[b][h],
                                 sem_idx[b])

        def drain_idx(b):
            for h in range(3):
                pltpu.make_async_copy(ih[h].at[pl.ds(0, T)], iv[b][h],
                                      sem_idx[b]).wait()
                pltpu.make_async_copy(dh[h].at[pl.ds(0, T)], dv[b][h],
                                      sem_idx[b]).wait()

        def fire_gather(b):
            for h in range(3):
                pltpu.async_copy(x_h.at[iv[b][h]], rr[b][h], sem_g[b])

        def drain_gather(b):
            for h in range(3):
                pltpu.make_async_copy(x_h.at[pl.ds(0, T)], rr[b][h],
                                      sem_g[b]).wait()

        def weights(b):
            for j in range(T // L):
                sl = pl.ds(j * L, L)
                q0 = 1.0 / (dv[b][0][sl] + EPS)
                q1 = 1.0 / (dv[b][1][sl] + EPS)
                q2 = 1.0 / (dv[b][2][sl] + EPS)
                nrm = q0 + q1 + q2
                wv[b][0][sl] = q0 / nrm
                wv[b][1][sl] = q1 / nrm
                wv[b][2][sl] = q2 / nrm

        def combine(b):
            def grp(g, c2):
                w0g = wv[b][0][pl.ds(g * L, L)]
                w1g = wv[b][1][pl.ds(g * L, L)]
                w2g = wv[b][2][pl.ds(g * L, L)]
                for j in range(L):
                    t = g * L + j
                    b0 = jnp.full((L,), w0g[j], jnp.float32)
                    b1 = jnp.full((L,), w1g[j], jnp.float32)
                    b2 = jnp.full((L,), w2g[j], jnp.float32)
                    for kk in range(C // L):
                        s = pl.ds(kk * L, L)
                        ov[b][t, s] = (b0 * rr[b][0][t, s]
                                       + b1 * rr[b][1][t, s]
                                       + b2 * rr[b][2][t, s])
                return c2
            lax.fori_loop(0, T // L, grp, 0)

        def fire_out(slot, b):
            pltpu.async_copy(ov[b], out_h.at[pl.ds(cbase(slot), T)],
                             sem_out[b])

        def drain_out(b):
            pltpu.make_async_copy(ov[b], out_h.at[pl.ds(0, T)],
                                  sem_out[b]).wait()

        def process(slot, b, first):
            weights(b)
            drain_gather(b)
            fire_idx(slot + 2, b)
            if not first:
                drain_out(b)
            combine(b)
            fire_out(slot, b)
            drain_idx(b)
            fire_gather(b)

        fire_idx(slot0, 0)
        fire_idx(slot0 + 1, 1)
        drain_idx(0)
        fire_gather(0)
        drain_idx(1)
        fire_gather(1)
        process(slot0, 0, True)
        process(slot0 + 1, 1, True)

        def pair(i2, carry):
            s = slot0 + i2 * 2
            process(s, 0, False)
            process(s + 1, 1, False)
            return carry

        lax.fori_loop(1, spw // 2, pair, 0)

        for b in range(2):
            drain_gather(b)
            drain_out(b)

    return k


def kernel(x, upsample_inds, upsample_dists):
    M = upsample_inds.shape[0]
    C = x.shape[1]
    inds = upsample_inds.astype(jnp.int32)
    d = upsample_dists.astype(jnp.float32)
    sc = _make_sc_kernel(M, C)
    return sc(x, inds[:, 0], inds[:, 1], inds[:, 2], d[:, 0], d[:, 1], d[:, 2])
